# baseline (speedup 1.0000x reference)
import jax
import jax.numpy as jnp
from jax import lax
from jax.experimental import pallas as pl
from jax.experimental.pallas import tpu as pltpu
from jax.experimental.pallas import tpu_sc as plsc

UNITS = 128
CHUNK = 128
NBUF = 4
LOOKAHEAD = 2
TC_BATCH = 2048
B_BLK = 128


def _tc_body(idx_ref, w1_ref, w2_ref, out_ref):
    sel = idx_ref[...][:, :, None]
    w1 = w1_ref[...][None]
    w2 = w2_ref[...][None]
    out_ref[...] = jnp.where(sel == 1, w1, 0.0) + jnp.where(sel == 2, w2, 0.0)


def _tc_part(idx, w1, w2):
    B, T = idx.shape
    U = w1.shape[1]
    return pl.pallas_call(
        _tc_body,
        grid=(B // B_BLK,),
        in_specs=[
            pl.BlockSpec((B_BLK, T), lambda i: (i, 0)),
            pl.BlockSpec((1, U), lambda i: (0, 0)),
            pl.BlockSpec((1, U), lambda i: (0, 0)),
        ],
        out_specs=pl.BlockSpec((B_BLK, T, U), lambda i: (i, 0, 0)),
        out_shape=jax.ShapeDtypeStruct((B, T, U), jnp.float32),
    )(idx, w1, w2)


def _sc_body(table_hbm, idx_hbm, out_hbm, table_sp, idx_all,
             rows0, rows1, rows2, rows3,
             sin0, sin1, sin2, sin3, sout0, sout1, sout2, sout3):
    rows = (rows0, rows1, rows2, rows3)
    sin = (sin0, sin1, sin2, sin3)
    sout = (sout0, sout1, sout2, sout3)

    info = plsc.get_sparse_core_info()
    nc, ns = info.num_cores, info.num_subcores
    nw = nc * ns
    cid = lax.axis_index("c")
    sid = lax.axis_index("s")
    wid = sid * nc + cid

    @pl.when(sid == 0)
    def _():
        pltpu.sync_copy(table_hbm, table_sp)

    plsc.subcore_barrier()

    n_rows = out_hbm.shape[0]
    rows_per_w = n_rows // nw
    n_chunks = rows_per_w // CHUNK
    n_groups = n_chunks // NBUF
    base = wid * rows_per_w

    pltpu.sync_copy(idx_hbm.at[pl.ds(base, rows_per_w)], idx_all)

    def gather(c, b):
        return pltpu.async_copy(table_sp.at[idx_all.at[pl.ds(c * CHUNK, CHUNK)]],
                                rows[b], sin[b])

    def wait_gather(b):
        pltpu.make_async_copy(table_sp.at[idx_all.at[pl.ds(0, CHUNK)]],
                              rows[b], sin[b]).wait()

    def put(c, b):
        return pltpu.async_copy(rows[b],
                                out_hbm.at[pl.ds(base + c * CHUNK, CHUNK)], sout[b])

    def wait_put(b):
        pltpu.make_async_copy(rows[b], out_hbm.at[pl.ds(base, CHUNK)], sout[b]).wait()

    for c in range(LOOKAHEAD):
        gather(c, c % NBUF)

    def group(g, carry):
        for db in range(NBUF):
            c = NBUF * g + db
            bg = (db + LOOKAHEAD) % NBUF

            @pl.when(jnp.logical_and(c + LOOKAHEAD < n_chunks,
                                     c + LOOKAHEAD >= NBUF))
            def _():
                wait_put(bg)

            @pl.when(c + LOOKAHEAD < n_chunks)
            def _():
                gather(c + LOOKAHEAD, bg)

            wait_gather(db)
            put(c, db)
        return carry

    lax.fori_loop(0, n_groups, group, 0)

    for b in range(NBUF):
        wait_put(b)


def _sc_part(idx_flat, w1, w2):
    U = w1.shape[1]
    n = idx_flat.shape[0]
    table = jnp.concatenate([jnp.zeros_like(w1), w1, w2], axis=0)
    mesh = plsc.VectorSubcoreMesh(core_axis_name="c", subcore_axis_name="s")
    rows_per_w = n // 32
    k = pl.kernel(
        _sc_body,
        out_type=jax.ShapeDtypeStruct((n, U), jnp.float32),
        mesh=mesh,
        scratch_types=[
            pltpu.VMEM_SHARED((3, U), jnp.float32),
            pltpu.VMEM((rows_per_w,), jnp.int32),
            pltpu.VMEM((CHUNK, U), jnp.float32),
            pltpu.VMEM((CHUNK, U), jnp.float32),
            pltpu.VMEM((CHUNK, U), jnp.float32),
            pltpu.VMEM((CHUNK, U), jnp.float32),
            pltpu.SemaphoreType.DMA,
            pltpu.SemaphoreType.DMA,
            pltpu.SemaphoreType.DMA,
            pltpu.SemaphoreType.DMA,
            pltpu.SemaphoreType.DMA,
            pltpu.SemaphoreType.DMA,
            pltpu.SemaphoreType.DMA,
            pltpu.SemaphoreType.DMA,
        ],
    )
    return k(table, idx_flat)


def kernel(inputs, w1, w2):
    B, T = inputs.shape
    U = w1.shape[1]
    idx = inputs.astype(jnp.int32)
    head = _tc_part(idx[:TC_BATCH], w1, w2)
    tail = _sc_part(idx[TC_BATCH:].reshape(-1), w1, w2)
    return jnp.concatenate([head, tail.reshape(B - TC_BATCH, T, U)], axis=0)


# speedup vs baseline: 1.9592x; 1.9592x over previous
import jax
import jax.numpy as jnp
from jax import lax
from jax.experimental import pallas as pl
from jax.experimental.pallas import tpu as pltpu
from jax.experimental.pallas import tpu_sc as plsc

UNITS = 128
CHUNK = 128
NBUF = 6
LOOKAHEAD = 3


def _sc_body(table_hbm, idx_hbm, out_hbm, table_sp, idx_all,
             rows0, rows1, rows2, rows3, rows4, rows5,
             sin0, sin1, sin2, sin3, sin4, sin5,
             sout0, sout1, sout2, sout3, sout4, sout5):
    rows = (rows0, rows1, rows2, rows3, rows4, rows5)
    sin = (sin0, sin1, sin2, sin3, sin4, sin5)
    sout = (sout0, sout1, sout2, sout3, sout4, sout5)

    info = plsc.get_sparse_core_info()
    nc, ns = info.num_cores, info.num_subcores
    nw = nc * ns
    cid = lax.axis_index("c")
    sid = lax.axis_index("s")
    wid = sid * nc + cid

    @pl.when(sid == 0)
    def _():
        pltpu.sync_copy(table_hbm, table_sp)

    plsc.subcore_barrier()

    n_rows = out_hbm.shape[0]
    rows_per_w = n_rows // nw
    n_chunks = rows_per_w // CHUNK
    n_groups = (n_chunks + NBUF - 1) // NBUF
    base = wid * rows_per_w

    pltpu.sync_copy(idx_hbm.at[pl.ds(base, rows_per_w)], idx_all)

    def gather(c, b):
        return pltpu.async_copy(table_sp.at[idx_all.at[pl.ds(c * CHUNK, CHUNK)]],
                                rows[b], sin[b])

    def wait_gather(b):
        pltpu.make_async_copy(table_sp.at[idx_all.at[pl.ds(0, CHUNK)]],
                              rows[b], sin[b]).wait()

    def put(c, b):
        return pltpu.async_copy(rows[b],
                                out_hbm.at[pl.ds(base + c * CHUNK, CHUNK)], sout[b])

    def wait_put(b):
        pltpu.make_async_copy(rows[b], out_hbm.at[pl.ds(base, CHUNK)], sout[b]).wait()

    for c in range(LOOKAHEAD):
        gather(c, c % NBUF)

    def group(g, carry):
        for db in range(NBUF):
            c = NBUF * g + db
            bg = (db + LOOKAHEAD) % NBUF

            @pl.when(jnp.logical_and(c + LOOKAHEAD < n_chunks,
                                     c + LOOKAHEAD >= NBUF))
            def _():
                wait_put(bg)

            @pl.when(c + LOOKAHEAD < n_chunks)
            def _():
                gather(c + LOOKAHEAD, bg)

            @pl.when(c < n_chunks)
            def _():
                wait_gather(db)
                put(c, db)
        return carry

    lax.fori_loop(0, n_groups, group, 0)

    for b in range(NBUF):
        wait_put(b)


def kernel(inputs, w1, w2):
    B, T = inputs.shape
    U = w1.shape[1]
    idx = inputs.reshape(-1).astype(jnp.int32)
    table = jnp.concatenate([jnp.zeros_like(w1), w1, w2], axis=0)
    mesh = plsc.VectorSubcoreMesh(core_axis_name="c", subcore_axis_name="s")
    rows_per_w = (B * T) // 32
    k = pl.kernel(
        _sc_body,
        out_type=jax.ShapeDtypeStruct((B * T, U), jnp.float32),
        mesh=mesh,
        scratch_types=(
            [pltpu.VMEM_SHARED((3, U), jnp.float32),
             pltpu.VMEM((rows_per_w,), jnp.int32)]
            + [pltpu.VMEM((CHUNK, U), jnp.float32)] * NBUF
            + [pltpu.SemaphoreType.DMA] * (2 * NBUF)
        ),
    )
    out = k(table, idx)
    return out.reshape(B, T, U)
